# baseline (speedup 1.0000x reference)
import functools
import numpy as np
import jax
import jax.numpy as jnp
from jax import lax
from jax.experimental import pallas as pl
from jax.experimental.pallas import tpu as pltpu
from jax.experimental.pallas import tpu_sc as plsc

N = 16384
BNUM = 8
KNN = 20
E = N * KNN
NEG = -1e30
F32 = jnp.float32


def _elu1(t):
    return jnp.where(t > 0, t + 1.0, jnp.exp(jnp.minimum(t, 0.0)))


def _dot(a, b):
    return jnp.dot(a, b, preferred_element_type=F32,
                   precision=lax.Precision.HIGHEST)


def _head_consts():
    Rm = np.zeros((32, 256), np.float32)
    Pm = np.zeros((256, 32), np.float32)
    Hm = np.zeros((32, 32), np.float32)
    for h in range(4):
        for d in range(8):
            for e in range(8):
                Rm[8 * h + d, 64 * h + 8 * d + e] = 1.0
                Pm[64 * h + 8 * d + e, 8 * h + e] = 1.0
    for c in range(32):
        for c2 in range(32):
            if c // 8 == c2 // 8:
                Hm[c, c2] = 1.0
    return Rm, Pm.T.copy(), Pm, Hm


_RM_N, _RV_N, _PM_N, _HM_N = _head_consts()


_KT = 256
_KCT = 256


def _knn_body(x8_ref, xT8_ref, brow_ref, bcol_ref, g8_ref, b8_ref,
              gT8_ref, bT8_ref, off_ref, out_ref, sp_ref, tp_ref, sd_ref):
    x8 = x8_ref[...]
    m8 = jnp.mean(x8, axis=0, keepdims=True)
    v8 = jnp.mean((x8 - m8) * (x8 - m8), axis=0, keepdims=True)
    sv8 = jnp.sqrt(v8 + 1e-5)
    s8 = g8_ref[...] / sv8
    t8 = b8_ref[...] - m8 * s8

    xT8 = xT8_ref[...]
    mT = jnp.mean(xT8, axis=1, keepdims=True)
    vT = jnp.mean((xT8 - mT) * (xT8 - mT), axis=1, keepdims=True)
    svT = jnp.sqrt(vT + 1e-5)
    gT = gT8_ref[...]
    bT = bT8_ref[...]

    s16 = jnp.concatenate([s8, jnp.zeros((1, 8), F32)], axis=1)
    t16 = jnp.concatenate([t8, jnp.zeros((1, 8), F32)], axis=1)
    sp_ref[...] = jnp.concatenate([s16] * 8, axis=1)
    tp_ref[...] = jnp.concatenate([t16] * 8, axis=1)

    iota_c = lax.broadcasted_iota(jnp.int32, (1, _KCT), 1).astype(F32)

    def row_tile(r, _):
        r0 = r * _KT
        rlast = r0 + _KT - 1
        b_first = jnp.int32(0)
        b_last = jnp.int32(0)
        for b in range(1, BNUM):
            b_first += jnp.where(off_ref[b] <= r0, 1, 0).astype(jnp.int32)
            b_last += jnp.where(off_ref[b] <= rlast, 1, 0).astype(jnp.int32)
        c_start = jnp.int32(0)
        c_end = jnp.int32(0)
        for b in range(BNUM):
            c_start += jnp.where(b_first == b, off_ref[b], 0).astype(jnp.int32)
            c_end += jnp.where(b_last == b, off_ref[b + 1], 0).astype(jnp.int32)
        ct0 = c_start // _KCT
        ct1 = (c_end + _KCT - 1) // _KCT

        xa = x8_ref[pl.ds(r0, _KT), :]
        An = (g8_ref[...] * (xa - m8)) / sv8 + b8_ref[...]
        browt = brow_ref[pl.ds(r0, _KT), :]

        def score_tile(ct, m):
            c0 = ct * _KCT
            xb = xT8_ref[:, pl.ds(c0, _KCT)]
            Bn = (gT * (xb - mT)) / svT + bT
            d2 = jnp.zeros((_KT, _KCT), F32)
            for c in range(4):
                diff = An[:, c:c + 1] - Bn[c:c + 1, :]
                d2 = d2 + diff * diff
            s = jnp.where(browt == bcol_ref[:, pl.ds(c0, _KCT)], -d2, NEG)
            sd_ref[:, pl.ds(c0, _KCT)] = s
            return jnp.maximum(m, jnp.max(s, axis=1, keepdims=True))
        m = lax.fori_loop(ct0, ct1, score_tile,
                          jnp.full((_KT, 1), NEG, F32))

        svec = jnp.zeros((_KT, 1), F32)
        evec = jnp.zeros((_KT, 1), F32)
        for b in range(BNUM):
            inb = (browt == float(b)).astype(F32)
            svec += inb * off_ref[b].astype(F32)
            evec += inb * off_ref[b + 1].astype(F32)

        cols = []
        cnt = jnp.zeros((_KT, 1), F32)
        for _t in range(KNN):
            def step_tile(ct, carry):
                idx, mnext = carry
                c0 = ct * _KCT
                tile = sd_ref[:, pl.ds(c0, _KCT)]
                eq = tile >= m
                li = jnp.min(jnp.where(eq, iota_c + c0.astype(F32), 3e9),
                             axis=1, keepdims=True)
                cleared = jnp.where(eq, NEG, tile)
                sd_ref[:, pl.ds(c0, _KCT)] = cleared
                return (jnp.minimum(idx, li),
                        jnp.maximum(mnext,
                                    jnp.max(cleared, axis=1, keepdims=True)))
            idx, mnext = lax.fori_loop(
                ct0, ct1, step_tile,
                (jnp.full((_KT, 1), 3e9, F32), jnp.full((_KT, 1), NEG, F32)))
            valid = m > (NEG * 0.5)
            filler = jnp.where(cnt < svec, cnt, evec + cnt - svec)
            cols.append(jnp.where(valid, idx, filler))
            cnt = cnt + jnp.where(valid, 0.0, 1.0)
            m = mnext

        cols.append(jnp.zeros((_KT, 32 - KNN), F32))
        out_ref[pl.ds(r0, _KT), :] = jnp.concatenate(cols, axis=1).astype(jnp.int32)
        return 0

    lax.fori_loop(0, N // _KT, row_tile, 0)


def _knn_call(x8, xT8, brow, bcol, g8, b8, gT8, bT8, offsets):
    return pl.pallas_call(
        _knn_body,
        out_shape=[jax.ShapeDtypeStruct((N, 32), jnp.int32),
                   jax.ShapeDtypeStruct((1, 128), F32),
                   jax.ShapeDtypeStruct((1, 128), F32)],
        in_specs=[pl.BlockSpec(memory_space=pltpu.VMEM)] * 8
        + [pl.BlockSpec(memory_space=pltpu.SMEM)],
        out_specs=[pl.BlockSpec(memory_space=pltpu.VMEM)] * 3,
        scratch_shapes=[pltpu.VMEM((_KT, N), F32)],
    )(x8, xT8, brow, bcol, g8, b8, gT8, bT8, offsets)


_NW = 32
_CH = 128
_PER_W = E // _NW


@functools.partial(
    pl.kernel,
    out_type=jax.ShapeDtypeStruct((E, 16), F32),
    mesh=plsc.VectorSubcoreMesh(core_axis_name="c", subcore_axis_name="s"),
    compiler_params=pltpu.CompilerParams(use_tc_tiling_on_sc=False),
    scratch_types=[
        pltpu.VMEM((_CH,), jnp.int32),
        pltpu.VMEM((_CH, 16), F32),
        pltpu.SemaphoreType.DMA,
    ],
)
def _sc_gather(table_hbm, idx_hbm, out_hbm, idx_v, rows_v, sem):
    wid = lax.axis_index("s") * 2 + lax.axis_index("c")
    base = wid * _PER_W

    def chunk(c, carry):
        off = base + c * _CH
        pltpu.sync_copy(idx_hbm.at[pl.ds(off, _CH)], idx_v)
        pltpu.async_copy(table_hbm.at[idx_v], rows_v, sem).wait()
        pltpu.sync_copy(rows_v, out_hbm.at[pl.ds(off, _CH)])
        return carry

    lax.fori_loop(0, _PER_W // _CH, chunk, 0)


_ST = 2048


def _stem_body(x8_ref, oh_ref, ohT_ref, w0_ref, b0_ref, lw_refs, out_ref, h_ref):
    ones_col = jnp.zeros((N, 1), F32) + 1.0
    counts = _dot(ohT_ref[...], ones_col)
    mmax = jnp.max(counts)
    mult = mmax - counts

    nt = N // _ST
    w0 = w0_ref[...]
    b0 = b0_ref[...]

    def emb_tile(i, carry):
        r0 = i * _ST
        h_ref[pl.ds(r0, _ST), :] = _dot(x8_ref[pl.ds(r0, _ST), :], w0) + b0
        return carry
    lax.fori_loop(0, nt, emb_tile, 0)
    hp = jnp.broadcast_to(b0, (BNUM, 32))

    rm = lw_refs[-4][...]
    rv = lw_refs[-3][...]
    pm = lw_refs[-2][...]
    hm = lw_refs[-1][...]

    for li in range(2):
        (wq, bq, wk, bk, wv, bv, wo, bo, n1g, n1b,
         f1w, f1b, f2w, f2b, n2g, n2b) = [r[...] for r in lw_refs[16 * li:16 * (li + 1)]]

        def acc_tile(i, carry):
            kv8, ks8 = carry
            r0 = i * _ST
            h = h_ref[pl.ds(r0, _ST), :]
            k = _elu1(_dot(h, wk) + bk)
            v = _dot(h, wv) + bv
            prod = _dot(k, rm) * _dot(v, rv)
            ohT = ohT_ref[:, pl.ds(r0, _ST)]
            return kv8 + _dot(ohT, prod), ks8 + _dot(ohT, k)
        kv8, ks8 = lax.fori_loop(
            0, nt, acc_tile,
            (jnp.zeros((BNUM, 256), F32), jnp.zeros((BNUM, 32), F32)))

        kp = _elu1(_dot(hp, wk) + bk)
        vp = _dot(hp, wv) + bv
        kv8 = kv8 + mult * (_dot(kp, rm) * _dot(vp, rv))
        ks8 = ks8 + mult * kp

        def ln(t, g, b):
            mu = jnp.mean(t, axis=-1, keepdims=True)
            va = jnp.mean((t - mu) * (t - mu), axis=-1, keepdims=True)
            return g * (t - mu) / jnp.sqrt(va + 1e-5) + b

        def transform(h, kvr, ksr):
            q = _elu1(_dot(h, wq) + bq)
            denom = _dot(q * ksr, hm) + 1e-6
            att = _dot(_dot(q, rm) * kvr, pm) / denom
            h1 = ln(h + _dot(att, wo) + bo, n1g, n1b)
            ff = _dot(jnp.maximum(_dot(h1, f1w) + f1b, 0.0), f2w) + f2b
            return ln(h1 + ff, n2g, n2b)

        def apply_tile(i, carry):
            r0 = i * _ST
            h = h_ref[pl.ds(r0, _ST), :]
            oh = oh_ref[pl.ds(r0, _ST), :]
            hn = transform(h, _dot(oh, kv8), _dot(oh, ks8))
            if li == 0:
                h_ref[pl.ds(r0, _ST), :] = hn
            else:
                out_ref[pl.ds(r0, _ST), :] = hn
            return carry
        lax.fori_loop(0, nt, apply_tile, 0)
        hp = transform(hp, kv8, ks8)


def _stem_call(x8, onehot, onehotT, w0, b0, layer_ws):
    nin = 5 + len(layer_ws)

    def body(*refs):
        _stem_body(refs[0], refs[1], refs[2], refs[3], refs[4],
                   refs[5:nin], refs[nin], refs[nin + 1])

    return pl.pallas_call(
        body,
        out_shape=jax.ShapeDtypeStruct((N, 32), F32),
        in_specs=[pl.BlockSpec(memory_space=pltpu.VMEM)] * nin,
        out_specs=pl.BlockSpec(memory_space=pltpu.VMEM),
        scratch_shapes=[pltpu.VMEM((N, 32), F32)],
    )(x8, onehot, onehotT, w0, b0, *layer_ws)


_TI = 2048
_TP = _TI // 8


def _edge_body(xp_ref, wg_ref, sp_ref, tp_ref, a1m_ref, b1m_ref, bb1_ref,
               w2p_ref, bb2_ref, g1_ref, be1_ref, g2_ref, be2_ref,
               fold_ref, tile8_ref, out_ref, acc_ref, aff_ref):
    p = pl.program_id(0)
    t = pl.program_id(1)
    s16 = sp_ref[...]
    t16 = tp_ref[...]
    inv_e = 1.0 / float(E)

    @pl.when((p == 0) & (t == 0))
    def _init():
        acc_ref[...] = jnp.zeros((4, 256), F32)
        aff_ref[...] = jnp.zeros((4, 256), F32)

    @pl.when((p == 1) & (t == 0))
    def _fin1():
        mu = _dot(acc_ref[0:1, :], fold_ref[...]) * inv_e
        var = _dot(acc_ref[1:2, :], fold_ref[...]) * inv_e - mu * mu
        a = g1_ref[...] / jnp.sqrt(var + 1e-5)
        c = be1_ref[...] - mu * a
        aff_ref[0:1, :] = _dot(a, tile8_ref[...])
        aff_ref[1:2, :] = _dot(c, tile8_ref[...])

    @pl.when((p == 2) & (t == 0))
    def _fin2():
        mu = _dot(acc_ref[2:3, :], fold_ref[...]) * inv_e
        var = _dot(acc_ref[3:4, :], fold_ref[...]) * inv_e - mu * mu
        a = g2_ref[...] / jnp.sqrt(var + 1e-5)
        c = be2_ref[...] - mu * a
        aff_ref[2:3, :] = _dot(a, tile8_ref[...])
        aff_ref[3:4, :] = _dot(c, tile8_ref[...])

    a1p = aff_ref[0:1, :]
    c1p = aff_ref[1:2, :]
    a2p = aff_ref[2:3, :]
    c2p = aff_ref[3:4, :]
    u = _dot(xp_ref[...] * s16 + t16, a1m_ref[...]) + bb1_ref[...]
    w2p = w2p_ref[...]
    bb2 = bb2_ref[...]
    o = jnp.full((_TP, 256), NEG, F32)
    s1 = jnp.zeros((1, 256), F32)
    q1 = jnp.zeros((1, 256), F32)
    s2 = jnp.zeros((1, 256), F32)
    q2 = jnp.zeros((1, 256), F32)
    for j in range(KNN):
        w = _dot(wg_ref[j, :, :] * s16 + t16, b1m_ref[...])
        h1 = u + w
        s1 = s1 + jnp.sum(h1, axis=0, keepdims=True)
        q1 = q1 + jnp.sum(h1 * h1, axis=0, keepdims=True)
        n1 = jnp.maximum(h1 * a1p + c1p, 0.0)
        h2 = _dot(n1, w2p) + bb2
        s2 = s2 + jnp.sum(h2, axis=0, keepdims=True)
        q2 = q2 + jnp.sum(h2 * h2, axis=0, keepdims=True)
        o = jnp.maximum(o, jnp.maximum(h2 * a2p + c2p, 0.0))

    @pl.when(p == 0)
    def _acc1():
        acc_ref[0:1, :] += s1
        acc_ref[1:2, :] += q1

    @pl.when(p == 1)
    def _acc2():
        acc_ref[2:3, :] += s2
        acc_ref[3:4, :] += q2

    out_ref[...] = o


def _edge_call(xp, wg, sp, tp, a1m, b1m, bb1, w2p, bb2, g1, be1, g2, be2,
               fold, tile8):
    full = pl.BlockSpec(memory_space=pltpu.VMEM)
    return pl.pallas_call(
        _edge_body,
        grid=(3, N // _TI),
        out_shape=jax.ShapeDtypeStruct((N // 8, 256), F32),
        in_specs=[
            pl.BlockSpec((_TP, 128), lambda p, t: (t, 0)),
            pl.BlockSpec((KNN, _TP, 128), lambda p, t: (0, t, 0)),
            full, full, full, full, full, full, full, full, full, full,
            full, full, full,
        ],
        out_specs=pl.BlockSpec((_TP, 256), lambda p, t: (t, 0)),
        scratch_shapes=[pltpu.VMEM((4, 256), F32), pltpu.VMEM((4, 256), F32)],
        compiler_params=pltpu.CompilerParams(
            dimension_semantics=("arbitrary", "arbitrary")),
    )(xp, wg, sp, tp, a1m, b1m, bb1, w2p, bb2, g1, be1, g2, be2, fold, tile8)


_CT = 2048


def _head_body(s_ref, p_ref, g_ref, w1s_ref, w1p_ref, w1g_ref, b1_ref,
               gg_ref, be_ref, w2_ref, b2_ref, out_ref):
    w1s = w1s_ref[...]
    w1p = w1p_ref[...]
    w1g = w1g_ref[...]
    b1 = b1_ref[...]
    nt = N // _CT
    inv_n = 1.0 / float(N)

    def zfun(r0):
        return (_dot(s_ref[pl.ds(r0, _CT), :], w1s)
                + _dot(p_ref[pl.ds(r0, _CT), :], w1p)
                + _dot(g_ref[pl.ds(r0, _CT), :], w1g) + b1)

    def pass1(i, carry):
        sz, qz = carry
        z = zfun(i * _CT)
        return (sz + jnp.sum(z, axis=0, keepdims=True),
                qz + jnp.sum(z * z, axis=0, keepdims=True))
    sz, qz = lax.fori_loop(0, nt, pass1,
                           (jnp.zeros((1, 128), F32), jnp.zeros((1, 128), F32)))
    mu = sz * inv_n
    var = qz * inv_n - mu * mu
    az = gg_ref[...] / jnp.sqrt(var + 1e-5)
    cz = be_ref[...] - mu * az
    w2 = w2_ref[...]
    b2 = b2_ref[...]

    def pass2(i, carry):
        r0 = i * _CT
        o = _dot(jnp.maximum(zfun(r0) * az + cz, 0.0), w2) + b2
        out_ref[pl.ds(r0, _CT), :] = o
        return carry
    lax.fori_loop(0, nt, pass2, 0)


def _head_call(outs, outp, gctx, w1s, w1p, w1g, b1, gg, be, w2, b2):
    return pl.pallas_call(
        _head_body,
        out_shape=jax.ShapeDtypeStruct((N, 8), F32),
        in_specs=[pl.BlockSpec(memory_space=pltpu.VMEM)] * 11,
        out_specs=pl.BlockSpec(memory_space=pltpu.VMEM),
    )(outs, outp, gctx, w1s, w1p, w1g, b1, gg, be, w2, b2)


def kernel(x, batch, params):
    p = params
    batch = batch.astype(jnp.int32)
    offsets = jnp.searchsorted(batch, jnp.arange(BNUM + 1)).astype(jnp.int32)

    def zc(a, w):
        return jnp.concatenate(
            [a, jnp.zeros((a.shape[0], w - a.shape[1]), F32)], axis=1)

    x8 = zc(x, 8)
    x16 = zc(x, 16)
    xT8 = jnp.concatenate([x.T, jnp.zeros((4, N), F32)], axis=0)
    bf = batch.astype(F32)
    brow = bf.reshape(N, 1)
    bcol = bf.reshape(1, N)
    onehot = (batch[:, None] == jnp.arange(BNUM)[None, :]).astype(F32)
    onehotT = onehot.T

    g3, b3 = p['bn3_g'], p['bn3_b']
    g1_, b1_ = p['bn1_g'], p['bn1_b']
    z4 = jnp.zeros((4,), F32)

    def row(v):
        return v.reshape(1, -1)

    def col(v):
        return v.reshape(-1, 1)

    g8_sp = row(jnp.concatenate([g3, jnp.zeros((5,), F32)]))
    b8_sp = row(jnp.concatenate([b3, jnp.zeros((5,), F32)]))
    g8_pe = row(jnp.concatenate([g3, g1_, z4]))
    b8_pe = row(jnp.concatenate([b3, b1_, z4]))

    nbr_s, sp_s, tp_s = _knn_call(x8, xT8, brow, bcol, g8_sp, b8_sp,
                                  col(g8_sp[0]), col(b8_sp[0]), offsets)
    nbr_p, sp_p, tp_p = _knn_call(x8, xT8, brow, bcol, g8_pe, b8_pe,
                                  col(g8_pe[0]), col(b8_pe[0]), offsets)

    idx_s = nbr_s[:, :KNN].T.reshape(-1)
    idx_p = nbr_p[:, :KNN].T.reshape(-1)
    xjk_s = _sc_gather(x16, idx_s).reshape(KNN, N // 8, 128)
    xjk_p = _sc_gather(x16, idx_p).reshape(KNN, N // 8, 128)

    w0 = jnp.concatenate([p['ge_W'].T[:3] + p['gp_W'].T,
                          p['ge_W'].T[3:4],
                          jnp.zeros((4, 32), F32)], axis=0)
    b0 = row(p['ge_b'] + p['gp_b'])
    layer_ws = []
    for lp in p['layers']:
        layer_ws += [lp['Wq'].T, row(lp['bq']), lp['Wk'].T, row(lp['bk']),
                     lp['Wv'].T, row(lp['bv']), lp['Wo'].T, row(lp['bo']),
                     row(lp['n1g']), row(lp['n1b']),
                     lp['f1W'].T, row(lp['f1b']), lp['f2W'].T, row(lp['f2b']),
                     row(lp['n2g']), row(lp['n2b'])]
    layer_ws += [jnp.asarray(_RM_N), jnp.asarray(_RV_N),
                 jnp.asarray(_PM_N), jnp.asarray(_HM_N)]
    gctx = _stem_call(x8, onehot, onehotT, w0, b0, layer_ws)

    xp = x16.reshape(N // 8, 128)
    eye8 = jnp.eye(8, dtype=F32)
    fold32 = jnp.kron(jnp.ones((8, 1), F32), jnp.eye(32, dtype=F32))
    tile32 = jnp.kron(jnp.ones((1, 8), F32), jnp.eye(32, dtype=F32))

    def edge(conv, xjk, cin, sp_, tp_):
        a1 = jnp.concatenate([(conv['W1'][:, :cin] - conv['W1'][:, cin:]).T,
                              jnp.zeros((16 - cin, 32), F32)], axis=0)
        b1w = jnp.concatenate([conv['W1'][:, cin:].T,
                               jnp.zeros((16 - cin, 32), F32)], axis=0)
        a1m = jnp.kron(eye8, a1)
        b1m = jnp.kron(eye8, b1w)
        bb1 = jnp.tile(row(conv['b1']), (1, 8))
        w2p = jnp.kron(eye8, conv['W2'].T)
        bb2 = jnp.tile(row(conv['b2']), (1, 8))
        outp_ = _edge_call(xp, xjk, sp_, tp_, a1m, b1m, bb1, w2p, bb2,
                           row(conv['g1']), row(conv['be1']),
                           row(conv['g2']), row(conv['be2']), fold32, tile32)
        return outp_.reshape(N, 32)

    out_s = edge(p['cs'], xjk_s, 3, sp_s, tp_s)
    out_p = edge(p['cp'], xjk_p, 4, sp_p, tp_p)

    cl = p['cl']
    w1s = cl['W1'][:, :32].T
    w1p = cl['W1'][:, 32:64].T
    w1g = cl['W1'][:, 64:96].T
    w2 = jnp.concatenate([cl['W2'].T, jnp.zeros((128, 7), F32)], axis=1)
    b2 = row(jnp.concatenate([cl['b2'], jnp.zeros((7,), F32)]))
    out = _head_call(out_s, out_p, gctx, w1s, w1p, w1g, row(cl['b1']),
                     row(cl['g']), row(cl['be']), w2, b2)
    return out[:, :1]


# speedup vs baseline: 1.4454x; 1.4454x over previous
import functools
import numpy as np
import jax
import jax.numpy as jnp
from jax import lax
from jax.experimental import pallas as pl
from jax.experimental.pallas import tpu as pltpu
from jax.experimental.pallas import tpu_sc as plsc

N = 16384
BNUM = 8
KNN = 20
E = N * KNN
NEG = -1e30
F32 = jnp.float32


def _elu1(t):
    return jnp.where(t > 0, t + 1.0, jnp.exp(jnp.minimum(t, 0.0)))


def _dot(a, b):
    return jnp.dot(a, b, preferred_element_type=F32,
                   precision=lax.Precision.HIGHEST)


def _head_consts():
    Rm = np.zeros((32, 256), np.float32)
    Pm = np.zeros((256, 32), np.float32)
    Hm = np.zeros((32, 32), np.float32)
    for h in range(4):
        for d in range(8):
            for e in range(8):
                Rm[8 * h + d, 64 * h + 8 * d + e] = 1.0
                Pm[64 * h + 8 * d + e, 8 * h + e] = 1.0
    for c in range(32):
        for c2 in range(32):
            if c // 8 == c2 // 8:
                Hm[c, c2] = 1.0
    return Rm, Pm.T.copy(), Pm, Hm


_RM_N, _RV_N, _PM_N, _HM_N = _head_consts()


_KT = 256
_KCT = 1024


def _knn_body(x8_ref, xT8_ref, brow_ref, bcol_ref, g8_ref, b8_ref,
              gT8_ref, bT8_ref, off_ref, out_ref, sp_ref, tp_ref, sd_ref):
    x8 = x8_ref[...]
    m8 = jnp.mean(x8, axis=0, keepdims=True)
    v8 = jnp.mean((x8 - m8) * (x8 - m8), axis=0, keepdims=True)
    sv8 = jnp.sqrt(v8 + 1e-5)
    s8 = g8_ref[...] / sv8
    t8 = b8_ref[...] - m8 * s8

    xT8 = xT8_ref[...]
    mT = jnp.mean(xT8, axis=1, keepdims=True)
    vT = jnp.mean((xT8 - mT) * (xT8 - mT), axis=1, keepdims=True)
    svT = jnp.sqrt(vT + 1e-5)
    gT = gT8_ref[...]
    bT = bT8_ref[...]

    s16 = jnp.concatenate([s8, jnp.zeros((1, 8), F32)], axis=1)
    t16 = jnp.concatenate([t8, jnp.zeros((1, 8), F32)], axis=1)
    sp_ref[...] = jnp.concatenate([s16] * 8, axis=1)
    tp_ref[...] = jnp.concatenate([t16] * 8, axis=1)

    iota_c = lax.broadcasted_iota(jnp.int32, (1, _KCT), 1).astype(F32)

    def row_tile(r, _):
        r0 = r * _KT
        rlast = r0 + _KT - 1
        b_first = jnp.int32(0)
        b_last = jnp.int32(0)
        for b in range(1, BNUM):
            b_first += jnp.where(off_ref[b] <= r0, 1, 0).astype(jnp.int32)
            b_last += jnp.where(off_ref[b] <= rlast, 1, 0).astype(jnp.int32)
        c_start = jnp.int32(0)
        c_end = jnp.int32(0)
        for b in range(BNUM):
            c_start += jnp.where(b_first == b, off_ref[b], 0).astype(jnp.int32)
            c_end += jnp.where(b_last == b, off_ref[b + 1], 0).astype(jnp.int32)
        ct0 = c_start // _KCT
        ct1 = (c_end + _KCT - 1) // _KCT

        xa = x8_ref[pl.ds(r0, _KT), :]
        An = (g8_ref[...] * (xa - m8)) / sv8 + b8_ref[...]
        browt = brow_ref[pl.ds(r0, _KT), :]

        def score_tile(ct, m):
            c0 = ct * _KCT
            xb = xT8_ref[:, pl.ds(c0, _KCT)]
            Bn = (gT * (xb - mT)) / svT + bT
            d2 = jnp.zeros((_KT, _KCT), F32)
            for c in range(4):
                diff = An[:, c:c + 1] - Bn[c:c + 1, :]
                d2 = d2 + diff * diff
            s = jnp.where(browt == bcol_ref[:, pl.ds(c0, _KCT)], -d2, NEG)
            sd_ref[:, pl.ds(c0, _KCT)] = s
            return jnp.maximum(m, jnp.max(s, axis=1, keepdims=True))
        m = lax.fori_loop(ct0, ct1, score_tile,
                          jnp.full((_KT, 1), NEG, F32))

        svec = jnp.zeros((_KT, 1), F32)
        evec = jnp.zeros((_KT, 1), F32)
        for b in range(BNUM):
            inb = (browt == float(b)).astype(F32)
            svec += inb * off_ref[b].astype(F32)
            evec += inb * off_ref[b + 1].astype(F32)

        cols = []
        cnt = jnp.zeros((_KT, 1), F32)
        for _t in range(KNN):
            def step_tile(ct, carry):
                idx, mnext = carry
                c0 = ct * _KCT
                tile = sd_ref[:, pl.ds(c0, _KCT)]
                eq = tile >= m
                li = jnp.min(jnp.where(eq, iota_c + c0.astype(F32), 3e9),
                             axis=1, keepdims=True)
                cleared = jnp.where(eq, NEG, tile)
                sd_ref[:, pl.ds(c0, _KCT)] = cleared
                return (jnp.minimum(idx, li),
                        jnp.maximum(mnext,
                                    jnp.max(cleared, axis=1, keepdims=True)))
            idx, mnext = lax.fori_loop(
                ct0, ct1, step_tile,
                (jnp.full((_KT, 1), 3e9, F32), jnp.full((_KT, 1), NEG, F32)))
            valid = m > (NEG * 0.5)
            filler = jnp.where(cnt < svec, cnt, evec + cnt - svec)
            cols.append(jnp.where(valid, idx, filler))
            cnt = cnt + jnp.where(valid, 0.0, 1.0)
            m = mnext

        cols.append(jnp.zeros((_KT, 32 - KNN), F32))
        out_ref[pl.ds(r0, _KT), :] = jnp.concatenate(cols, axis=1).astype(jnp.int32)
        return 0

    lax.fori_loop(0, N // _KT, row_tile, 0)


def _knn_call(x8, xT8, brow, bcol, g8, b8, gT8, bT8, offsets):
    return pl.pallas_call(
        _knn_body,
        out_shape=[jax.ShapeDtypeStruct((N, 32), jnp.int32),
                   jax.ShapeDtypeStruct((1, 128), F32),
                   jax.ShapeDtypeStruct((1, 128), F32)],
        in_specs=[pl.BlockSpec(memory_space=pltpu.VMEM)] * 8
        + [pl.BlockSpec(memory_space=pltpu.SMEM)],
        out_specs=[pl.BlockSpec(memory_space=pltpu.VMEM)] * 3,
        scratch_shapes=[pltpu.VMEM((_KT, N), F32)],
    )(x8, xT8, brow, bcol, g8, b8, gT8, bT8, offsets)


_NW = 32
_CH = 128
_PER_W = E // _NW


@functools.partial(
    pl.kernel,
    out_type=jax.ShapeDtypeStruct((E, 16), F32),
    mesh=plsc.VectorSubcoreMesh(core_axis_name="c", subcore_axis_name="s"),
    compiler_params=pltpu.CompilerParams(use_tc_tiling_on_sc=False),
    scratch_types=[
        pltpu.VMEM((_CH,), jnp.int32),
        pltpu.VMEM((_CH, 16), F32),
        pltpu.SemaphoreType.DMA,
    ],
)
def _sc_gather(table_hbm, idx_hbm, out_hbm, idx_v, rows_v, sem):
    wid = lax.axis_index("s") * 2 + lax.axis_index("c")
    base = wid * _PER_W

    def chunk(c, carry):
        off = base + c * _CH
        pltpu.sync_copy(idx_hbm.at[pl.ds(off, _CH)], idx_v)
        pltpu.async_copy(table_hbm.at[idx_v], rows_v, sem).wait()
        pltpu.sync_copy(rows_v, out_hbm.at[pl.ds(off, _CH)])
        return carry

    lax.fori_loop(0, _PER_W // _CH, chunk, 0)


_ST = 2048


def _stem_body(x8_ref, oh_ref, ohT_ref, w0_ref, b0_ref, lw_refs, out_ref, h_ref):
    ones_col = jnp.zeros((N, 1), F32) + 1.0
    counts = _dot(ohT_ref[...], ones_col)
    mmax = jnp.max(counts)
    mult = mmax - counts

    nt = N // _ST
    w0 = w0_ref[...]
    b0 = b0_ref[...]

    def emb_tile(i, carry):
        r0 = i * _ST
        h_ref[pl.ds(r0, _ST), :] = _dot(x8_ref[pl.ds(r0, _ST), :], w0) + b0
        return carry
    lax.fori_loop(0, nt, emb_tile, 0)
    hp = jnp.broadcast_to(b0, (BNUM, 32))

    rm = lw_refs[-4][...]
    rv = lw_refs[-3][...]
    pm = lw_refs[-2][...]
    hm = lw_refs[-1][...]

    for li in range(2):
        (wq, bq, wk, bk, wv, bv, wo, bo, n1g, n1b,
         f1w, f1b, f2w, f2b, n2g, n2b) = [r[...] for r in lw_refs[16 * li:16 * (li + 1)]]

        def acc_tile(i, carry):
            kv8, ks8 = carry
            r0 = i * _ST
            h = h_ref[pl.ds(r0, _ST), :]
            k = _elu1(_dot(h, wk) + bk)
            v = _dot(h, wv) + bv
            prod = _dot(k, rm) * _dot(v, rv)
            ohT = ohT_ref[:, pl.ds(r0, _ST)]
            return kv8 + _dot(ohT, prod), ks8 + _dot(ohT, k)
        kv8, ks8 = lax.fori_loop(
            0, nt, acc_tile,
            (jnp.zeros((BNUM, 256), F32), jnp.zeros((BNUM, 32), F32)))

        kp = _elu1(_dot(hp, wk) + bk)
        vp = _dot(hp, wv) + bv
        kv8 = kv8 + mult * (_dot(kp, rm) * _dot(vp, rv))
        ks8 = ks8 + mult * kp

        def ln(t, g, b):
            mu = jnp.mean(t, axis=-1, keepdims=True)
            va = jnp.mean((t - mu) * (t - mu), axis=-1, keepdims=True)
            return g * (t - mu) / jnp.sqrt(va + 1e-5) + b

        def transform(h, kvr, ksr):
            q = _elu1(_dot(h, wq) + bq)
            denom = _dot(q * ksr, hm) + 1e-6
            att = _dot(_dot(q, rm) * kvr, pm) / denom
            h1 = ln(h + _dot(att, wo) + bo, n1g, n1b)
            ff = _dot(jnp.maximum(_dot(h1, f1w) + f1b, 0.0), f2w) + f2b
            return ln(h1 + ff, n2g, n2b)

        def apply_tile(i, carry):
            r0 = i * _ST
            h = h_ref[pl.ds(r0, _ST), :]
            oh = oh_ref[pl.ds(r0, _ST), :]
            hn = transform(h, _dot(oh, kv8), _dot(oh, ks8))
            if li == 0:
                h_ref[pl.ds(r0, _ST), :] = hn
            else:
                out_ref[pl.ds(r0, _ST), :] = hn
            return carry
        lax.fori_loop(0, nt, apply_tile, 0)
        hp = transform(hp, kv8, ks8)


def _stem_call(x8, onehot, onehotT, w0, b0, layer_ws):
    nin = 5 + len(layer_ws)

    def body(*refs):
        _stem_body(refs[0], refs[1], refs[2], refs[3], refs[4],
                   refs[5:nin], refs[nin], refs[nin + 1])

    return pl.pallas_call(
        body,
        out_shape=jax.ShapeDtypeStruct((N, 32), F32),
        in_specs=[pl.BlockSpec(memory_space=pltpu.VMEM)] * nin,
        out_specs=pl.BlockSpec(memory_space=pltpu.VMEM),
        scratch_shapes=[pltpu.VMEM((N, 32), F32)],
    )(x8, onehot, onehotT, w0, b0, *layer_ws)


_TI = 2048
_TP = _TI // 8


def _edge_body(xp_ref, wg_ref, sp_ref, tp_ref, a1m_ref, b1m_ref, bb1_ref,
               w2p_ref, bb2_ref, g1_ref, be1_ref, g2_ref, be2_ref,
               fold_ref, tile8_ref, out_ref, acc_ref, aff_ref):
    p = pl.program_id(0)
    t = pl.program_id(1)
    s16 = sp_ref[...]
    t16 = tp_ref[...]
    inv_e = 1.0 / float(E)

    @pl.when((p == 0) & (t == 0))
    def _init():
        acc_ref[...] = jnp.zeros((4, 256), F32)
        aff_ref[...] = jnp.zeros((4, 256), F32)

    @pl.when((p == 1) & (t == 0))
    def _fin1():
        mu = _dot(acc_ref[0:1, :], fold_ref[...]) * inv_e
        var = _dot(acc_ref[1:2, :], fold_ref[...]) * inv_e - mu * mu
        a = g1_ref[...] / jnp.sqrt(var + 1e-5)
        c = be1_ref[...] - mu * a
        aff_ref[0:1, :] = _dot(a, tile8_ref[...])
        aff_ref[1:2, :] = _dot(c, tile8_ref[...])

    @pl.when((p == 2) & (t == 0))
    def _fin2():
        mu = _dot(acc_ref[2:3, :], fold_ref[...]) * inv_e
        var = _dot(acc_ref[3:4, :], fold_ref[...]) * inv_e - mu * mu
        a = g2_ref[...] / jnp.sqrt(var + 1e-5)
        c = be2_ref[...] - mu * a
        aff_ref[2:3, :] = _dot(a, tile8_ref[...])
        aff_ref[3:4, :] = _dot(c, tile8_ref[...])

    a1p = aff_ref[0:1, :]
    c1p = aff_ref[1:2, :]
    a2p = aff_ref[2:3, :]
    c2p = aff_ref[3:4, :]
    u = _dot(xp_ref[...] * s16 + t16, a1m_ref[...]) + bb1_ref[...]
    w2p = w2p_ref[...]
    bb2 = bb2_ref[...]
    o = jnp.full((_TP, 256), NEG, F32)
    s1 = jnp.zeros((1, 256), F32)
    q1 = jnp.zeros((1, 256), F32)
    s2 = jnp.zeros((1, 256), F32)
    q2 = jnp.zeros((1, 256), F32)
    for j in range(KNN):
        w = _dot(wg_ref[j, :, :] * s16 + t16, b1m_ref[...])
        h1 = u + w
        s1 = s1 + jnp.sum(h1, axis=0, keepdims=True)
        q1 = q1 + jnp.sum(h1 * h1, axis=0, keepdims=True)
        n1 = jnp.maximum(h1 * a1p + c1p, 0.0)
        h2 = _dot(n1, w2p) + bb2
        s2 = s2 + jnp.sum(h2, axis=0, keepdims=True)
        q2 = q2 + jnp.sum(h2 * h2, axis=0, keepdims=True)
        o = jnp.maximum(o, jnp.maximum(h2 * a2p + c2p, 0.0))

    @pl.when(p == 0)
    def _acc1():
        acc_ref[0:1, :] += s1
        acc_ref[1:2, :] += q1

    @pl.when(p == 1)
    def _acc2():
        acc_ref[2:3, :] += s2
        acc_ref[3:4, :] += q2

    out_ref[...] = o


def _edge_call(xp, wg, sp, tp, a1m, b1m, bb1, w2p, bb2, g1, be1, g2, be2,
               fold, tile8):
    full = pl.BlockSpec(memory_space=pltpu.VMEM)
    return pl.pallas_call(
        _edge_body,
        grid=(3, N // _TI),
        out_shape=jax.ShapeDtypeStruct((N // 8, 256), F32),
        in_specs=[
            pl.BlockSpec((_TP, 128), lambda p, t: (t, 0)),
            pl.BlockSpec((KNN, _TP, 128), lambda p, t: (0, t, 0)),
            full, full, full, full, full, full, full, full, full, full,
            full, full, full,
        ],
        out_specs=pl.BlockSpec((_TP, 256), lambda p, t: (t, 0)),
        scratch_shapes=[pltpu.VMEM((4, 256), F32), pltpu.VMEM((4, 256), F32)],
        compiler_params=pltpu.CompilerParams(
            dimension_semantics=("arbitrary", "arbitrary")),
    )(xp, wg, sp, tp, a1m, b1m, bb1, w2p, bb2, g1, be1, g2, be2, fold, tile8)


_CT = 2048


def _head_body(s_ref, p_ref, g_ref, w1s_ref, w1p_ref, w1g_ref, b1_ref,
               gg_ref, be_ref, w2_ref, b2_ref, out_ref):
    w1s = w1s_ref[...]
    w1p = w1p_ref[...]
    w1g = w1g_ref[...]
    b1 = b1_ref[...]
    nt = N // _CT
    inv_n = 1.0 / float(N)

    def zfun(r0):
        return (_dot(s_ref[pl.ds(r0, _CT), :], w1s)
                + _dot(p_ref[pl.ds(r0, _CT), :], w1p)
                + _dot(g_ref[pl.ds(r0, _CT), :], w1g) + b1)

    def pass1(i, carry):
        sz, qz = carry
        z = zfun(i * _CT)
        return (sz + jnp.sum(z, axis=0, keepdims=True),
                qz + jnp.sum(z * z, axis=0, keepdims=True))
    sz, qz = lax.fori_loop(0, nt, pass1,
                           (jnp.zeros((1, 128), F32), jnp.zeros((1, 128), F32)))
    mu = sz * inv_n
    var = qz * inv_n - mu * mu
    az = gg_ref[...] / jnp.sqrt(var + 1e-5)
    cz = be_ref[...] - mu * az
    w2 = w2_ref[...]
    b2 = b2_ref[...]

    def pass2(i, carry):
        r0 = i * _CT
        o = _dot(jnp.maximum(zfun(r0) * az + cz, 0.0), w2) + b2
        out_ref[pl.ds(r0, _CT), :] = o
        return carry
    lax.fori_loop(0, nt, pass2, 0)


def _head_call(outs, outp, gctx, w1s, w1p, w1g, b1, gg, be, w2, b2):
    return pl.pallas_call(
        _head_body,
        out_shape=jax.ShapeDtypeStruct((N, 8), F32),
        in_specs=[pl.BlockSpec(memory_space=pltpu.VMEM)] * 11,
        out_specs=pl.BlockSpec(memory_space=pltpu.VMEM),
    )(outs, outp, gctx, w1s, w1p, w1g, b1, gg, be, w2, b2)


def kernel(x, batch, params):
    p = params
    batch = batch.astype(jnp.int32)
    offsets = jnp.searchsorted(batch, jnp.arange(BNUM + 1)).astype(jnp.int32)

    def zc(a, w):
        return jnp.concatenate(
            [a, jnp.zeros((a.shape[0], w - a.shape[1]), F32)], axis=1)

    x8 = zc(x, 8)
    x16 = zc(x, 16)
    xT8 = jnp.concatenate([x.T, jnp.zeros((4, N), F32)], axis=0)
    bf = batch.astype(F32)
    brow = bf.reshape(N, 1)
    bcol = bf.reshape(1, N)
    onehot = (batch[:, None] == jnp.arange(BNUM)[None, :]).astype(F32)
    onehotT = onehot.T

    g3, b3 = p['bn3_g'], p['bn3_b']
    g1_, b1_ = p['bn1_g'], p['bn1_b']
    z4 = jnp.zeros((4,), F32)

    def row(v):
        return v.reshape(1, -1)

    def col(v):
        return v.reshape(-1, 1)

    g8_sp = row(jnp.concatenate([g3, jnp.zeros((5,), F32)]))
    b8_sp = row(jnp.concatenate([b3, jnp.zeros((5,), F32)]))
    g8_pe = row(jnp.concatenate([g3, g1_, z4]))
    b8_pe = row(jnp.concatenate([b3, b1_, z4]))

    nbr_s, sp_s, tp_s = _knn_call(x8, xT8, brow, bcol, g8_sp, b8_sp,
                                  col(g8_sp[0]), col(b8_sp[0]), offsets)
    nbr_p, sp_p, tp_p = _knn_call(x8, xT8, brow, bcol, g8_pe, b8_pe,
                                  col(g8_pe[0]), col(b8_pe[0]), offsets)

    idx_s = nbr_s[:, :KNN].T.reshape(-1)
    idx_p = nbr_p[:, :KNN].T.reshape(-1)
    xjk_s = _sc_gather(x16, idx_s).reshape(KNN, N // 8, 128)
    xjk_p = _sc_gather(x16, idx_p).reshape(KNN, N // 8, 128)

    w0 = jnp.concatenate([p['ge_W'].T[:3] + p['gp_W'].T,
                          p['ge_W'].T[3:4],
                          jnp.zeros((4, 32), F32)], axis=0)
    b0 = row(p['ge_b'] + p['gp_b'])
    layer_ws = []
    for lp in p['layers']:
        layer_ws += [lp['Wq'].T, row(lp['bq']), lp['Wk'].T, row(lp['bk']),
                     lp['Wv'].T, row(lp['bv']), lp['Wo'].T, row(lp['bo']),
                     row(lp['n1g']), row(lp['n1b']),
                     lp['f1W'].T, row(lp['f1b']), lp['f2W'].T, row(lp['f2b']),
                     row(lp['n2g']), row(lp['n2b'])]
    layer_ws += [jnp.asarray(_RM_N), jnp.asarray(_RV_N),
                 jnp.asarray(_PM_N), jnp.asarray(_HM_N)]
    gctx = _stem_call(x8, onehot, onehotT, w0, b0, layer_ws)

    xp = x16.reshape(N // 8, 128)
    eye8 = jnp.eye(8, dtype=F32)
    fold32 = jnp.kron(jnp.ones((8, 1), F32), jnp.eye(32, dtype=F32))
    tile32 = jnp.kron(jnp.ones((1, 8), F32), jnp.eye(32, dtype=F32))

    def edge(conv, xjk, cin, sp_, tp_):
        a1 = jnp.concatenate([(conv['W1'][:, :cin] - conv['W1'][:, cin:]).T,
                              jnp.zeros((16 - cin, 32), F32)], axis=0)
        b1w = jnp.concatenate([conv['W1'][:, cin:].T,
                               jnp.zeros((16 - cin, 32), F32)], axis=0)
        a1m = jnp.kron(eye8, a1)
        b1m = jnp.kron(eye8, b1w)
        bb1 = jnp.tile(row(conv['b1']), (1, 8))
        w2p = jnp.kron(eye8, conv['W2'].T)
        bb2 = jnp.tile(row(conv['b2']), (1, 8))
        outp_ = _edge_call(xp, xjk, sp_, tp_, a1m, b1m, bb1, w2p, bb2,
                           row(conv['g1']), row(conv['be1']),
                           row(conv['g2']), row(conv['be2']), fold32, tile32)
        return outp_.reshape(N, 32)

    out_s = edge(p['cs'], xjk_s, 3, sp_s, tp_s)
    out_p = edge(p['cp'], xjk_p, 4, sp_p, tp_p)

    cl = p['cl']
    w1s = cl['W1'][:, :32].T
    w1p = cl['W1'][:, 32:64].T
    w1g = cl['W1'][:, 64:96].T
    w2 = jnp.concatenate([cl['W2'].T, jnp.zeros((128, 7), F32)], axis=1)
    b2 = row(jnp.concatenate([cl['b2'], jnp.zeros((7,), F32)]))
    out = _head_call(out_s, out_p, gctx, w1s, w1p, w1g, row(cl['b1']),
                     row(cl['g']), row(cl['be']), w2, b2)
    return out[:, :1]
